# Initial kernel scaffold; baseline (speedup 1.0000x reference)
import jax
import jax.numpy as jnp
from jax.experimental import pallas as pl


def kernel(x, router_logits, w1, w3, w2):
    raise NotImplementedError("write your pallas kernel here")


# speedup vs baseline: 3.4972x; 3.4972x over previous
import functools

import jax
import jax.numpy as jnp
from jax.experimental import pallas as pl
from jax.experimental.pallas import tpu as pltpu


def _moe_gemm_body(eid_ref, xg_ref, w1_ref, w3_ref, w2_ref, out_ref):
    j = pl.program_id(1)
    x = xg_ref[...]
    w1b = w1_ref[0]
    w3b = w3_ref[0]
    w2b = w2_ref[0]
    g = jax.lax.dot_general(x, w1b, (((1,), (1,)), ((), ())),
                            preferred_element_type=jnp.float32)
    u = jax.lax.dot_general(x, w3b, (((1,), (1,)), ((), ())),
                            preferred_element_type=jnp.float32)
    h = (g * jax.nn.sigmoid(g)) * u
    y = jax.lax.dot_general(h, w2b, (((1,), (1,)), ((), ())),
                            preferred_element_type=jnp.float32)

    @pl.when(j == 0)
    def _():
        out_ref[...] = y

    @pl.when(j > 0)
    def _():
        out_ref[...] += y


def _grouped_gemm(tile_eid, xg, w1, w3, w2, *, tile_t, tile_f, interpret=False):
    nt = xg.shape[0] // tile_t
    nf = w1.shape[1] // tile_f
    d = xg.shape[1]
    grid_spec = pltpu.PrefetchScalarGridSpec(
        num_scalar_prefetch=1,
        grid=(nt, nf),
        in_specs=[
            pl.BlockSpec((tile_t, d), lambda i, j, eids: (i, 0)),
            pl.BlockSpec((1, tile_f, d), lambda i, j, eids: (eids[i], j, 0)),
            pl.BlockSpec((1, tile_f, d), lambda i, j, eids: (eids[i], j, 0)),
            pl.BlockSpec((1, d, tile_f), lambda i, j, eids: (eids[i], 0, j)),
        ],
        out_specs=pl.BlockSpec((tile_t, d), lambda i, j, eids: (i, 0)),
    )
    return pl.pallas_call(
        _moe_gemm_body,
        grid_spec=grid_spec,
        out_shape=jax.ShapeDtypeStruct((nt * tile_t, d), jnp.float32),
        compiler_params=pltpu.CompilerParams(
            dimension_semantics=("arbitrary", "arbitrary"),
        ),
        interpret=interpret,
    )(tile_eid, xg, w1, w3, w2)


def _kernel_impl(x, router_logits, w1, w3, w2, *, interpret=False):
    t, d = x.shape
    e, f, _ = w1.shape
    tile_t = min(512, t)
    tile_f = min(512, f)
    nt = t // tile_t + (e - 1)
    pcap = nt * tile_t

    eid = jnp.argmax(router_logits, axis=-1)
    top = jnp.max(router_logits, axis=-1)
    scale = jax.nn.sigmoid(top)
    xs = x * scale[:, None]

    onehot = (eid[:, None] == jnp.arange(e)[None, :]).astype(jnp.int32)
    counts = jnp.sum(onehot, axis=0)
    pc = ((counts + tile_t - 1) // tile_t) * tile_t
    ends = jnp.cumsum(pc)
    starts = ends - pc
    rank = jnp.cumsum(onehot, axis=0) - 1
    rank_t = jnp.take_along_axis(rank, eid[:, None], axis=1)[:, 0]
    pos = (starts[eid] + rank_t).astype(jnp.int32)

    tile_start = jnp.arange(nt) * tile_t
    tile_eid = jnp.searchsorted(ends, tile_start, side="right")
    tile_eid = jnp.minimum(tile_eid, e - 1).astype(jnp.int32)

    xg = jnp.zeros((pcap, d), x.dtype).at[pos].set(xs)

    y = _grouped_gemm(tile_eid, xg, w1, w3, w2,
                      tile_t=tile_t, tile_f=tile_f, interpret=interpret)

    return y[pos]


def kernel(x, router_logits, w1, w3, w2):
    return _kernel_impl(x, router_logits, w1, w3, w2)
